# baseline (speedup 1.0000x reference)
import functools

import jax
import jax.numpy as jnp
from jax import lax
from jax.experimental import pallas as pl
from jax.experimental.pallas import tpu as pltpu
from jax.experimental.pallas import tpu_sc as plsc

N = 10000
E = 320000
D = 128

NC = 2
NS = 16
NW = NC * NS
N_PAD = 10240
EP = E // NW
CHUNK = 128
FULL_CHUNKS = EP // CHUNK
CPT = 80
E_PAD = NW * CPT * CHUNK
N_ACC = 10112
ACC_PER_TILE = N_ACC // NS

_MESH = plsc.VectorSubcoreMesh(core_axis_name="c", subcore_axis_name="s")


@functools.partial(
    pl.kernel,
    out_type=jax.ShapeDtypeStruct((NC, 80, 128), jnp.float32),
    mesh=_MESH,
    scratch_types=[
        pltpu.VMEM((EP,), jnp.int32),
        pltpu.VMEM((80, 128), jnp.float32),
        pltpu.VMEM((80,), jnp.int32),
        pltpu.VMEM_SHARED((80, 128), jnp.float32),
    ],
    compiler_params=pltpu.CompilerParams(needs_layout_passes=False),
)
def _deg_kernel(dst_hbm, zeros_hbm, out_hbm, idx_buf, deg_loc,
                row_idx, deg_shared):
    cid = lax.axis_index("c")
    sid = lax.axis_index("s")
    wid = sid * NC + cid

    @pl.when(sid == 0)
    def _():
        pltpu.sync_copy(zeros_hbm, deg_shared)

    pltpu.sync_copy(zeros_hbm, deg_loc)

    iota = lax.iota(jnp.int32, 16)
    for k in range(5):
        row_idx[pl.ds(k * 16, 16)] = iota + (k * 16)

    plsc.subcore_barrier()

    ones = jnp.ones((16,), jnp.float32)
    base = pl.multiple_of(wid * EP, 8)
    pltpu.sync_copy(dst_hbm.at[pl.ds(base, EP)], idx_buf)

    def body(k, _):
        for j in range(5):
            idx16 = idx_buf[pl.ds((k * 5 + j) * 16, 16)]
            row = jnp.right_shift(idx16, 7)
            col = jnp.bitwise_and(idx16, 127)
            plsc.addupdate_scatter(deg_loc, [row, col], ones)
        return 0

    lax.fori_loop(0, EP // 80, body, 0)

    pltpu.sync_copy(deg_loc, deg_shared.at[row_idx], add=True)
    plsc.subcore_barrier()

    @pl.when(sid == 0)
    def _():
        pltpu.sync_copy(deg_shared, out_hbm.at[cid])


def _mm_body(x_ref, w_ref, deg_ref, z_ref, norm_ref):
    deg = deg_ref[0] + deg_ref[1]
    norm = lax.rsqrt(jnp.maximum(deg, 1.0))
    norm_ref[...] = norm
    y = lax.dot_general(x_ref[...], w_ref[...],
                        (((1,), (1,)), ((), ())),
                        preferred_element_type=jnp.float32)
    z_ref[...] = y * norm[:N]


_mm_kernel = pl.pallas_call(
    _mm_body,
    out_shape=(
        jax.ShapeDtypeStruct((N, D), jnp.float32),
        jax.ShapeDtypeStruct((N_PAD, 1), jnp.float32),
    ),
)


@functools.partial(
    pl.kernel,
    out_type=jax.ShapeDtypeStruct((NC, N_ACC, 128), jnp.float32),
    mesh=_MESH,
    scratch_types=[
        pltpu.VMEM((CPT, CHUNK), jnp.int32),
        pltpu.VMEM((CHUNK,), jnp.int32),
        pltpu.VMEM((CHUNK,), jnp.int32),
        pltpu.VMEM((CHUNK, 128), jnp.float32),
        pltpu.VMEM((CHUNK, 128), jnp.float32),
        pltpu.VMEM_SHARED((N_ACC, 128), jnp.float32),
        pltpu.SemaphoreType.DMA,
        pltpu.SemaphoreType.DMA,
        pltpu.SemaphoreType.DMA,
        pltpu.SemaphoreType.DMA,
    ],
)
def _agg_kernel(z_hbm, src_hbm, dst_hbm, zeros_hbm, out_hbm,
                src_loc, dst_a, dst_b, rows_a, rows_b, acc,
                sem_a, sem_b, semi_a, semi_b):
    cid = lax.axis_index("c")
    sid = lax.axis_index("s")
    wid = sid * NC + cid

    pltpu.sync_copy(zeros_hbm, acc.at[pl.ds(sid * ACC_PER_TILE, ACC_PER_TILE)])
    pltpu.sync_copy(src_hbm.at[wid], src_loc)
    plsc.subcore_barrier()

    def start(k, dbuf, rbuf, semi, sem):
        kk = jnp.minimum(k, CPT - 1)
        off = pl.multiple_of((wid * CPT + kk) * CHUNK, 8)
        pltpu.sync_copy(dst_hbm.at[pl.ds(off, CHUNK)], dbuf)
        pltpu.async_copy(z_hbm.at[src_loc.at[kk]], rbuf, sem)

    def wait_rows(rbuf, sem):
        pltpu.make_async_copy(z_hbm.at[pl.ds(0, CHUNK)], rbuf, sem).wait()

    def wait_idx(dbuf, semi):
        pltpu.make_async_copy(dst_hbm.at[pl.ds(0, CHUNK)], dbuf, semi).wait()

    start(0, dst_a, rows_a, semi_a, sem_a)
    start(1, dst_b, rows_b, semi_b, sem_b)

    def body(p, _):
        wait_rows(rows_a, sem_a)
        pltpu.sync_copy(rows_a, acc.at[dst_a], add=True)
        start(2 * p + 2, dst_a, rows_a, semi_a, sem_a)
        wait_rows(rows_b, sem_b)
        pltpu.sync_copy(rows_b, acc.at[dst_b], add=True)
        start(2 * p + 3, dst_b, rows_b, semi_b, sem_b)
        return 0

    lax.fori_loop(0, CPT // 2, body, 0)
    wait_rows(rows_a, sem_a)
    wait_rows(rows_b, sem_b)

    plsc.subcore_barrier()
    pltpu.sync_copy(acc.at[pl.ds(sid * ACC_PER_TILE, ACC_PER_TILE)],
                    out_hbm.at[cid, pl.ds(sid * ACC_PER_TILE, ACC_PER_TILE)])


def _fin_body(p_ref, norm_ref, h_ref):
    u = p_ref[0, :N, :] + p_ref[1, :N, :]
    h_ref[...] = u * norm_ref[:N]


_fin_kernel = pl.pallas_call(
    _fin_body,
    out_shape=jax.ShapeDtypeStruct((N, D), jnp.float32),
)


def kernel(x, edge_index, W):
    src = edge_index[0]
    dst = edge_index[1]
    zeros_deg = jnp.zeros((80, 128), jnp.float32)

    pad = E_PAD - E
    src3 = jnp.concatenate([src, jnp.zeros((pad,), jnp.int32)]
                           ).reshape(NW, CPT, CHUNK)
    trash_ids = N + (jnp.arange(pad, dtype=jnp.int32) % (N_ACC - N))
    dst_flat = jnp.concatenate([dst, trash_ids])
    zeros_acc = jnp.zeros((ACC_PER_TILE, 128), jnp.float32)

    deg_rows = _deg_kernel(dst, zeros_deg)
    deg_col = deg_rows.reshape(NC, N_PAD, 1)
    z, norm_col = _mm_kernel(x, W, deg_col)
    parts = _agg_kernel(z, src3, dst_flat, zeros_acc)
    return _fin_kernel(parts, norm_col)


# speedup vs baseline: 1.0208x; 1.0208x over previous
import functools

import jax
import jax.numpy as jnp
from jax import lax
from jax.experimental import pallas as pl
from jax.experimental.pallas import tpu as pltpu
from jax.experimental.pallas import tpu_sc as plsc

N = 10000
E = 320000
D = 128

NC = 2
NS = 16
NW = NC * NS
N_PAD = 10240
EP = E // NW
CHUNK = 128
FULL_CHUNKS = EP // CHUNK
CPT = 80
E_PAD = NW * CPT * CHUNK
N_ACC = 10112
ACC_PER_TILE = N_ACC // NS

_MESH = plsc.VectorSubcoreMesh(core_axis_name="c", subcore_axis_name="s")


@functools.partial(
    pl.kernel,
    out_type=jax.ShapeDtypeStruct((NC, 80, 128), jnp.float32),
    mesh=_MESH,
    scratch_types=[
        pltpu.VMEM((EP,), jnp.int32),
        pltpu.VMEM((80, 128), jnp.float32),
        pltpu.VMEM((80,), jnp.int32),
        pltpu.VMEM_SHARED((80, 128), jnp.float32),
    ],
    compiler_params=pltpu.CompilerParams(needs_layout_passes=False),
)
def _deg_kernel(dst_hbm, zeros_hbm, out_hbm, idx_buf, deg_loc,
                row_idx, deg_shared):
    cid = lax.axis_index("c")
    sid = lax.axis_index("s")
    wid = sid * NC + cid

    @pl.when(sid == 0)
    def _():
        pltpu.sync_copy(zeros_hbm, deg_shared)

    pltpu.sync_copy(zeros_hbm, deg_loc)

    iota = lax.iota(jnp.int32, 16)
    for k in range(5):
        row_idx[pl.ds(k * 16, 16)] = iota + (k * 16)

    plsc.subcore_barrier()

    ones = jnp.ones((16,), jnp.float32)
    base = pl.multiple_of(wid * EP, 8)
    pltpu.sync_copy(dst_hbm.at[pl.ds(base, EP)], idx_buf)

    def body(k, _):
        for j in range(5):
            idx16 = idx_buf[pl.ds((k * 5 + j) * 16, 16)]
            row = jnp.right_shift(idx16, 7)
            col = jnp.bitwise_and(idx16, 127)
            plsc.addupdate_scatter(deg_loc, [row, col], ones)
        return 0

    lax.fori_loop(0, EP // 80, body, 0)

    pltpu.sync_copy(deg_loc, deg_shared.at[row_idx], add=True)
    plsc.subcore_barrier()

    @pl.when(sid == 0)
    def _():
        pltpu.sync_copy(deg_shared, out_hbm.at[cid])


def _mm_body(x_ref, w_ref, deg_ref, z_ref, norm_ref):
    deg = deg_ref[0] + deg_ref[1]
    norm = lax.rsqrt(jnp.maximum(deg, 1.0))
    norm_ref[...] = norm
    y = lax.dot_general(x_ref[...], w_ref[...],
                        (((1,), (1,)), ((), ())),
                        preferred_element_type=jnp.float32)
    z_ref[...] = y * norm[:N]


_mm_kernel = pl.pallas_call(
    _mm_body,
    out_shape=(
        jax.ShapeDtypeStruct((N, D), jnp.float32),
        jax.ShapeDtypeStruct((N_PAD, 1), jnp.float32),
    ),
)


@functools.partial(
    pl.kernel,
    out_type=jax.ShapeDtypeStruct((NC, N_ACC, 128), jnp.float32),
    mesh=_MESH,
    scratch_types=[
        pltpu.VMEM((CHUNK,), jnp.int32),
        pltpu.VMEM((CHUNK,), jnp.int32),
        pltpu.VMEM((CHUNK,), jnp.int32),
        pltpu.VMEM((CHUNK,), jnp.int32),
        pltpu.VMEM((CHUNK, 128), jnp.float32),
        pltpu.VMEM((CHUNK, 128), jnp.float32),
        pltpu.VMEM_SHARED((N_ACC, 128), jnp.float32),
        pltpu.SemaphoreType.DMA,
        pltpu.SemaphoreType.DMA,
        pltpu.SemaphoreType.DMA,
        pltpu.SemaphoreType.DMA,
    ],
)
def _agg_kernel(z_hbm, src_hbm, dst_hbm, zeros_hbm, out_hbm,
                src_a, src_b, dst_a, dst_b, rows_a, rows_b, acc,
                sem_a, sem_b, semi_a, semi_b):
    cid = lax.axis_index("c")
    sid = lax.axis_index("s")
    wid = sid * NC + cid

    pltpu.sync_copy(zeros_hbm, acc.at[pl.ds(sid * ACC_PER_TILE, ACC_PER_TILE)])
    plsc.subcore_barrier()

    def start(k, sbuf, dbuf, rbuf, semi, sem):
        kk = jnp.minimum(k, CPT - 1)
        off = pl.multiple_of((wid * CPT + kk) * CHUNK, 8)
        pltpu.sync_copy(src_hbm.at[pl.ds(off, CHUNK)], sbuf)
        pltpu.sync_copy(dst_hbm.at[pl.ds(off, CHUNK)], dbuf)
        pltpu.async_copy(z_hbm.at[sbuf], rbuf, sem)

    def wait_rows(rbuf, sem):
        pltpu.make_async_copy(z_hbm.at[pl.ds(0, CHUNK)], rbuf, sem).wait()

    def wait_idx(dbuf, semi):
        pltpu.make_async_copy(dst_hbm.at[pl.ds(0, CHUNK)], dbuf, semi).wait()

    start(0, src_a, dst_a, rows_a, semi_a, sem_a)
    start(1, src_b, dst_b, rows_b, semi_b, sem_b)

    def body(p, _):
        wait_rows(rows_a, sem_a)
        pltpu.sync_copy(rows_a, acc.at[dst_a], add=True)
        start(2 * p + 2, src_a, dst_a, rows_a, semi_a, sem_a)
        wait_rows(rows_b, sem_b)
        pltpu.sync_copy(rows_b, acc.at[dst_b], add=True)
        start(2 * p + 3, src_b, dst_b, rows_b, semi_b, sem_b)
        return 0

    lax.fori_loop(0, CPT // 2, body, 0)
    wait_rows(rows_a, sem_a)
    wait_rows(rows_b, sem_b)

    plsc.subcore_barrier()
    pltpu.sync_copy(acc.at[pl.ds(sid * ACC_PER_TILE, ACC_PER_TILE)],
                    out_hbm.at[cid, pl.ds(sid * ACC_PER_TILE, ACC_PER_TILE)])


def _fin_body(p_ref, norm_ref, h_ref):
    u = p_ref[0, :N, :] + p_ref[1, :N, :]
    h_ref[...] = u * norm_ref[:N]


_fin_kernel = pl.pallas_call(
    _fin_body,
    out_shape=jax.ShapeDtypeStruct((N, D), jnp.float32),
)


def kernel(x, edge_index, W):
    src = edge_index[0]
    dst = edge_index[1]
    zeros_deg = jnp.zeros((80, 128), jnp.float32)

    pad = E_PAD - E
    src_flat = jnp.concatenate([src, jnp.zeros((pad,), jnp.int32)])
    trash_ids = N + (jnp.arange(pad, dtype=jnp.int32) % (N_ACC - N))
    dst_flat = jnp.concatenate([dst, trash_ids])
    zeros_acc = jnp.zeros((ACC_PER_TILE, 128), jnp.float32)

    deg_rows = _deg_kernel(dst, zeros_deg)
    deg_col = deg_rows.reshape(NC, N_PAD, 1)
    z, norm_col = _mm_kernel(x, W, deg_col)
    parts = _agg_kernel(z, src_flat, dst_flat, zeros_acc)
    return _fin_kernel(parts, norm_col)


# speedup vs baseline: 2.4671x; 2.4169x over previous
import functools

import jax
import jax.numpy as jnp
from jax import lax
from jax.experimental import pallas as pl
from jax.experimental.pallas import tpu as pltpu
from jax.experimental.pallas import tpu_sc as plsc

N = 10000
E = 320000
D = 128

NC = 2
NS = 16
NW = NC * NS
N_PAD = 10240
EP = E // NW
CHUNK = 128
FULL_CHUNKS = EP // CHUNK
CPT = 80
E_PAD = NW * CPT * CHUNK
N_ACC = 10112
ACC_PER_TILE = N_ACC // NS

_MESH = plsc.VectorSubcoreMesh(core_axis_name="c", subcore_axis_name="s")


@functools.partial(
    pl.kernel,
    out_type=jax.ShapeDtypeStruct((NC, 80, 128), jnp.float32),
    mesh=_MESH,
    scratch_types=[
        pltpu.VMEM((EP,), jnp.int32),
        pltpu.VMEM((80, 128), jnp.float32),
        pltpu.VMEM((80,), jnp.int32),
        pltpu.VMEM_SHARED((80, 128), jnp.float32),
    ],
    compiler_params=pltpu.CompilerParams(needs_layout_passes=False),
)
def _deg_kernel(dst_hbm, zeros_hbm, out_hbm, idx_buf, deg_loc,
                row_idx, deg_shared):
    cid = lax.axis_index("c")
    sid = lax.axis_index("s")
    wid = sid * NC + cid

    @pl.when(sid == 0)
    def _():
        pltpu.sync_copy(zeros_hbm, deg_shared)

    pltpu.sync_copy(zeros_hbm, deg_loc)

    iota = lax.iota(jnp.int32, 16)
    for k in range(5):
        row_idx[pl.ds(k * 16, 16)] = iota + (k * 16)

    plsc.subcore_barrier()

    ones = jnp.ones((16,), jnp.float32)
    base = pl.multiple_of(wid * EP, 8)
    pltpu.sync_copy(dst_hbm.at[pl.ds(base, EP)], idx_buf)

    def body(k, _):
        for j in range(5):
            idx16 = idx_buf[pl.ds((k * 5 + j) * 16, 16)]
            row = jnp.right_shift(idx16, 7)
            col = jnp.bitwise_and(idx16, 127)
            plsc.addupdate_scatter(deg_loc, [row, col], ones)
        return 0

    lax.fori_loop(0, EP // 80, body, 0)

    pltpu.sync_copy(deg_loc, deg_shared.at[row_idx], add=True)
    plsc.subcore_barrier()

    @pl.when(sid == 0)
    def _():
        pltpu.sync_copy(deg_shared, out_hbm.at[cid])


def _mm_body(x_ref, w_ref, deg_ref, z_ref, norm_ref):
    deg = deg_ref[0] + deg_ref[1]
    norm = lax.rsqrt(jnp.maximum(deg, 1.0))
    norm_ref[...] = norm
    y = lax.dot_general(x_ref[...], w_ref[...],
                        (((1,), (1,)), ((), ())),
                        preferred_element_type=jnp.float32)
    z_ref[...] = y * norm[:N]


_mm_kernel = pl.pallas_call(
    _mm_body,
    out_shape=(
        jax.ShapeDtypeStruct((N, D), jnp.float32),
        jax.ShapeDtypeStruct((N_PAD, 1), jnp.float32),
    ),
)


@functools.partial(
    pl.kernel,
    out_type=jax.ShapeDtypeStruct((NC, N_ACC, 128), jnp.float32),
    mesh=_MESH,
    scratch_types=[
        pltpu.VMEM((CHUNK,), jnp.int32),
        pltpu.VMEM((CHUNK,), jnp.int32),
        pltpu.VMEM((CHUNK,), jnp.int32),
        pltpu.VMEM((CHUNK,), jnp.int32),
        pltpu.VMEM((CHUNK, 128), jnp.float32),
        pltpu.VMEM((CHUNK, 128), jnp.float32),
        pltpu.VMEM_SHARED((N_ACC, 128), jnp.float32),
        pltpu.SemaphoreType.DMA,
        pltpu.SemaphoreType.DMA,
        pltpu.SemaphoreType.DMA,
        pltpu.SemaphoreType.DMA,
    ],
)
def _agg_kernel(z_hbm, src_hbm, dst_hbm, zeros_hbm, out_hbm,
                src_a, src_b, dst_a, dst_b, rows_a, rows_b, acc,
                sem_a, sem_b, semi_a, semi_b):
    cid = lax.axis_index("c")
    sid = lax.axis_index("s")
    wid = sid * NC + cid

    pltpu.sync_copy(zeros_hbm, acc.at[pl.ds(sid * ACC_PER_TILE, ACC_PER_TILE)])
    plsc.subcore_barrier()

    def start(k, sbuf, dbuf, rbuf, semi, sem):
        kk = jnp.minimum(k, CPT - 1)
        off = pl.multiple_of((wid * CPT + kk) * CHUNK, 8)
        pltpu.sync_copy(src_hbm.at[pl.ds(off, CHUNK)], sbuf)
        pltpu.sync_copy(dst_hbm.at[pl.ds(off, CHUNK)], dbuf)
        pltpu.async_copy(z_hbm.at[sbuf], rbuf, sem)

    def wait_rows(rbuf, sem):
        pltpu.make_async_copy(z_hbm.at[pl.ds(0, CHUNK)], rbuf, sem).wait()

    def wait_idx(dbuf, semi):
        pltpu.make_async_copy(dst_hbm.at[pl.ds(0, CHUNK)], dbuf, semi).wait()

    start(0, src_a, dst_a, rows_a, semi_a, sem_a)
    start(1, src_b, dst_b, rows_b, semi_b, sem_b)

    def body(p, _):
        wait_rows(rows_a, sem_a)
        pltpu.sync_copy(rows_a, acc.at[dst_a], add=True)
        start(2 * p + 2, src_a, dst_a, rows_a, semi_a, sem_a)
        wait_rows(rows_b, sem_b)
        pltpu.sync_copy(rows_b, acc.at[dst_b], add=True)
        start(2 * p + 3, src_b, dst_b, rows_b, semi_b, sem_b)
        return 0

    lax.fori_loop(0, CPT // 2, body, 0)
    wait_rows(rows_a, sem_a)
    wait_rows(rows_b, sem_b)

    plsc.subcore_barrier()
    pltpu.sync_copy(acc.at[pl.ds(sid * ACC_PER_TILE, ACC_PER_TILE)],
                    out_hbm.at[cid, pl.ds(sid * ACC_PER_TILE, ACC_PER_TILE)])


def _fin_body(p_ref, norm_ref, h_ref):
    u = p_ref[0, :N, :] + p_ref[1, :N, :]
    h_ref[...] = u * norm_ref[:N]


_fin_kernel = pl.pallas_call(
    _fin_body,
    out_shape=jax.ShapeDtypeStruct((N, D), jnp.float32),
)


def kernel(x, edge_index, W):
    src = edge_index[0]
    dst = edge_index[1]
    zeros_deg = jnp.zeros((80, 128), jnp.float32)

    pad = E_PAD - E
    pad_iota = jnp.arange(pad, dtype=jnp.int32)
    src_flat = jnp.concatenate([src, pad_iota % N])
    dst_flat = jnp.concatenate([dst, N + (pad_iota % (N_ACC - N))])
    zeros_acc = jnp.zeros((ACC_PER_TILE, 128), jnp.float32)

    deg_rows = _deg_kernel(dst, zeros_deg)
    deg_col = deg_rows.reshape(NC, N_PAD, 1)
    z, norm_col = _mm_kernel(x, W, deg_col)
    parts = _agg_kernel(z, src_flat, dst_flat, zeros_acc)
    return _fin_kernel(parts, norm_col)


# speedup vs baseline: 3.0537x; 1.2378x over previous
import functools

import jax
import jax.numpy as jnp
from jax import lax
from jax.experimental import pallas as pl
from jax.experimental.pallas import tpu as pltpu
from jax.experimental.pallas import tpu_sc as plsc

N = 10000
E = 320000
D = 128

NC = 2
NS = 16
NW = NC * NS
N_PAD = 10240
EP = E // NW
CHUNK = 128
FULL_CHUNKS = EP // CHUNK
CPT = 80
E_PAD = NW * CPT * CHUNK
N_ACC = 10112
ACC_PER_TILE = N_ACC // NS

_MESH = plsc.VectorSubcoreMesh(core_axis_name="c", subcore_axis_name="s")


@functools.partial(
    pl.kernel,
    out_type=jax.ShapeDtypeStruct((NC, 80, 128), jnp.float32),
    mesh=_MESH,
    scratch_types=[
        pltpu.VMEM((EP,), jnp.int32),
        pltpu.VMEM((80, 128), jnp.float32),
        pltpu.VMEM((80,), jnp.int32),
        pltpu.VMEM_SHARED((80, 128), jnp.float32),
    ],
    compiler_params=pltpu.CompilerParams(needs_layout_passes=False),
)
def _deg_kernel(dst_hbm, zeros_hbm, out_hbm, idx_buf, deg_loc,
                row_idx, deg_shared):
    cid = lax.axis_index("c")
    sid = lax.axis_index("s")
    wid = sid * NC + cid

    @pl.when(sid == 0)
    def _():
        pltpu.sync_copy(zeros_hbm, deg_shared)

    pltpu.sync_copy(zeros_hbm, deg_loc)

    iota = lax.iota(jnp.int32, 16)
    for k in range(5):
        row_idx[pl.ds(k * 16, 16)] = iota + (k * 16)

    plsc.subcore_barrier()

    ones = jnp.ones((16,), jnp.float32)
    base = pl.multiple_of(wid * EP, 8)
    pltpu.sync_copy(dst_hbm.at[pl.ds(base, EP)], idx_buf)

    def body(k, _):
        for j in range(5):
            idx16 = idx_buf[pl.ds((k * 5 + j) * 16, 16)]
            row = jnp.right_shift(idx16, 7)
            col = jnp.bitwise_and(idx16, 127)
            plsc.addupdate_scatter(deg_loc, [row, col], ones)
        return 0

    lax.fori_loop(0, EP // 80, body, 0)

    pltpu.sync_copy(deg_loc, deg_shared.at[row_idx], add=True)
    plsc.subcore_barrier()

    @pl.when(sid == 0)
    def _():
        pltpu.sync_copy(deg_shared, out_hbm.at[cid])


def _mm_body(x_ref, w_ref, deg_ref, z_ref, norm_ref):
    deg = deg_ref[0] + deg_ref[1]
    norm = lax.rsqrt(jnp.maximum(deg, 1.0))
    norm_ref[...] = norm
    y = lax.dot_general(x_ref[...], w_ref[...],
                        (((1,), (1,)), ((), ())),
                        preferred_element_type=jnp.float32)
    z_ref[...] = y * norm[:N]


_mm_kernel = pl.pallas_call(
    _mm_body,
    out_shape=(
        jax.ShapeDtypeStruct((N, D), jnp.float32),
        jax.ShapeDtypeStruct((N_PAD, 1), jnp.float32),
    ),
)


@functools.partial(
    pl.kernel,
    out_type=jax.ShapeDtypeStruct((NC, N_ACC, 128), jnp.float32),
    mesh=_MESH,
    scratch_types=[
        pltpu.VMEM((CPT, CHUNK), jnp.int32),
        pltpu.VMEM((CHUNK,), jnp.int32),
        pltpu.VMEM((CHUNK,), jnp.int32),
        pltpu.VMEM((CHUNK, 128), jnp.float32),
        pltpu.VMEM((CHUNK, 128), jnp.float32),
        pltpu.VMEM_SHARED((N_ACC, 128), jnp.float32),
        pltpu.SemaphoreType.DMA,
        pltpu.SemaphoreType.DMA,
        pltpu.SemaphoreType.DMA,
        pltpu.SemaphoreType.DMA,
    ],
)
def _agg_kernel(z_hbm, src_hbm, dst_hbm, zeros_hbm, out_hbm,
                src_loc, dst_a, dst_b, rows_a, rows_b, acc,
                sem_a, sem_b, semi_a, semi_b):
    cid = lax.axis_index("c")
    sid = lax.axis_index("s")
    wid = sid * NC + cid

    pltpu.sync_copy(zeros_hbm, acc.at[pl.ds(sid * ACC_PER_TILE, ACC_PER_TILE)])
    pltpu.sync_copy(src_hbm.at[wid], src_loc)
    plsc.subcore_barrier()

    def start(k, dbuf, rbuf, semi, sem):
        kk = jnp.minimum(k, CPT - 1)
        off = pl.multiple_of((wid * CPT + kk) * CHUNK, 8)
        pltpu.async_copy(dst_hbm.at[pl.ds(off, CHUNK)], dbuf, semi)
        pltpu.async_copy(z_hbm.at[src_loc.at[kk]], rbuf, sem)

    def wait_rows(rbuf, sem):
        pltpu.make_async_copy(z_hbm.at[pl.ds(0, CHUNK)], rbuf, sem).wait()

    def wait_idx(dbuf, semi):
        pltpu.make_async_copy(dst_hbm.at[pl.ds(0, CHUNK)], dbuf, semi).wait()

    start(0, dst_a, rows_a, semi_a, sem_a)
    start(1, dst_b, rows_b, semi_b, sem_b)

    def body(p, _):
        wait_rows(rows_a, sem_a)
        wait_idx(dst_a, semi_a)
        pltpu.sync_copy(rows_a, acc.at[dst_a], add=True)
        start(2 * p + 2, dst_a, rows_a, semi_a, sem_a)
        wait_rows(rows_b, sem_b)
        wait_idx(dst_b, semi_b)
        pltpu.sync_copy(rows_b, acc.at[dst_b], add=True)
        start(2 * p + 3, dst_b, rows_b, semi_b, sem_b)
        return 0

    lax.fori_loop(0, CPT // 2, body, 0)
    wait_rows(rows_a, sem_a)
    wait_rows(rows_b, sem_b)
    wait_idx(dst_a, semi_a)
    wait_idx(dst_b, semi_b)

    plsc.subcore_barrier()
    pltpu.sync_copy(acc.at[pl.ds(sid * ACC_PER_TILE, ACC_PER_TILE)],
                    out_hbm.at[cid, pl.ds(sid * ACC_PER_TILE, ACC_PER_TILE)])


def _fin_body(p_ref, norm_ref, h_ref):
    u = p_ref[0, :N, :] + p_ref[1, :N, :]
    h_ref[...] = u * norm_ref[:N]


_fin_kernel = pl.pallas_call(
    _fin_body,
    out_shape=jax.ShapeDtypeStruct((N, D), jnp.float32),
)


def kernel(x, edge_index, W):
    src = edge_index[0]
    dst = edge_index[1]
    zeros_deg = jnp.zeros((80, 128), jnp.float32)

    pad = E_PAD - E
    pad_iota = jnp.arange(pad, dtype=jnp.int32)
    src3 = jnp.concatenate([src, pad_iota % N]).reshape(NW, CPT, CHUNK)
    dst_flat = jnp.concatenate([dst, N + (pad_iota % (N_ACC - N))])
    zeros_acc = jnp.zeros((ACC_PER_TILE, 128), jnp.float32)

    deg_rows = _deg_kernel(dst, zeros_deg)
    deg_col = deg_rows.reshape(NC, N_PAD, 1)
    z, norm_col = _mm_kernel(x, W, deg_col)
    parts = _agg_kernel(z, src3, dst_flat, zeros_acc)
    return _fin_kernel(parts, norm_col)
